# baseline (speedup 1.0000x reference)
import functools

import jax
import jax.numpy as jnp
from jax import lax
from jax.experimental import pallas as pl
from jax.experimental.pallas import tpu as pltpu
from jax.experimental.pallas import tpu_sc as plsc

NUM_EMB = 1000000
DIM = 64
BATCH = 16384
SEQ = 50
NC = 2
NS = 16
NW = NC * NS
BT = BATCH // 128
NTASK = SEQ * BT
TPW = NTASK // NW
NBUF = 2
G_OUTER = TPW // NBUF
BLK = 8 * 8 * 128
OUT_FLAT = SEQ * 8 * 128 * 8 * 128


def _build():
    mesh = plsc.VectorSubcoreMesh(core_axis_name="c", subcore_axis_name="s")

    @functools.partial(
        pl.kernel,
        mesh=mesh,
        out_type=jax.ShapeDtypeStruct((OUT_FLAT,), jnp.float32),
        scratch_types=[
            pltpu.VMEM((3, BATCH), jnp.int32),
            [pltpu.VMEM((128, DIM), jnp.float32) for _ in range(NBUF)],
            [pltpu.VMEM((BLK,), jnp.float32) for _ in range(NBUF)],
            [pltpu.SemaphoreType.DMA for _ in range(NBUF)],
            [pltpu.SemaphoreType.DMA for _ in range(NBUF)],
        ],
        compiler_params=pltpu.CompilerParams(
            use_tc_tiling_on_sc=False, needs_layout_passes=False
        ),
    )
    def gather_kernel(ids_hbm, table_hbm, out_hbm, ids_v, gbufs, tbufs,
                      fsems, wsems):
        wid = lax.axis_index("s") * NC + lax.axis_index("c")
        base_p = wid * TPW
        t0 = base_p // 128

        for j in range(3):
            row = jnp.minimum(t0 + j, SEQ - 1)
            pltpu.sync_copy(ids_hbm.at[pl.ds(row, 1)], ids_v.at[pl.ds(j, 1)])

        dvecs = [(lax.iota(jnp.int32, 16) + dg * 16) * 128 for dg in range(4)]

        def fill(k, b):
            p = base_p + k
            t = p >> 7
            tc = p & 127
            pltpu.async_copy(
                table_hbm.at[ids_v.at[t - t0, pl.ds(tc * 128, 128)]],
                gbufs[b],
                fsems[b],
            )

        def wait_fill(b):
            pltpu.make_async_copy(
                table_hbm.at[pl.ds(0, 128)], gbufs[b], fsems[b]
            ).wait()

        def transpose(b):
            def body(cc, carry):
                for dg in range(4):
                    v = gbufs[b][cc, pl.ds(dg * 16, 16)]
                    plsc.store_scatter(tbufs[b], [dvecs[dg] + cc], v)
                return carry

            lax.fori_loop(0, 128, body, 0)

        def drain(k, b):
            p = base_p + k
            t = p >> 7
            tc = p & 127
            for tr in range(8):
                pltpu.async_copy(
                    tbufs[b].at[pl.ds(tr * 1024, 1024)],
                    out_hbm.at[pl.ds((t * 8 + tr) * 131072 + tc * 1024, 1024)],
                    wsems[b],
                )

        def wait_drain(b):
            pltpu.make_async_copy(
                tbufs[b], out_hbm.at[pl.ds(0, BLK)], wsems[b]
            ).wait()

        for b in range(NBUF):
            fill(b, b)
        for b in range(NBUF):
            wait_fill(b)
            transpose(b)
            drain(b, b)
            fill(b + NBUF, b)

        def outer(g, carry):
            for b in range(NBUF):
                k = g * NBUF + b
                wait_fill(b)
                wait_drain(b)
                transpose(b)
                drain(k, b)
                fill(k + NBUF, b)
            return carry

        lax.fori_loop(1, G_OUTER - 1, outer, 0)

        for b in range(NBUF):
            k = (G_OUTER - 1) * NBUF + b
            wait_fill(b)
            wait_drain(b)
            transpose(b)
            drain(k, b)
        for b in range(NBUF):
            wait_drain(b)

    return gather_kernel


_gather = _build()


def kernel(token_ids, EmbeddingLayer):
    x = _gather(token_ids.astype(jnp.int32).T, EmbeddingLayer)
    x5 = x.reshape(SEQ, 8, 128, 8, 128)
    return x5.transpose(2, 4, 0, 1, 3).reshape(BATCH, SEQ, DIM)
